# baseline (speedup 1.0000x reference)
import functools

import jax
import jax.numpy as jnp
from jax import lax
from jax.experimental import pallas as pl
from jax.experimental.pallas import tpu as pltpu
from jax.experimental.pallas import tpu_sc as plsc


def _sc_select_gather(idx_t, tab_t, B, V, row_lo, n_rows):
    R = tab_t.shape[0]
    F = idx_t.shape[0]
    d_per_f = R // F
    n_workers = 32
    rows_per_w = n_rows // n_workers
    och = 4096
    n_och = B // och
    n_buf = 2
    unroll = 8
    mesh = plsc.VectorSubcoreMesh(core_axis_name="c", subcore_axis_name="s")

    @functools.partial(
        pl.kernel,
        out_type=jax.ShapeDtypeStruct((n_rows, B), jnp.float32),
        mesh=mesh,
        scratch_types=[
            pltpu.VMEM((B,), jnp.int32),
            pltpu.VMEM((V,), jnp.float32),
            pltpu.VMEM((n_buf, och), jnp.float32),
            pltpu.SemaphoreType.DMA,
        ],
        compiler_params=pltpu.CompilerParams(use_tc_tiling_on_sc=True,
                                             needs_layout_passes=False),
    )
    def sel_kernel(idx_hbm, tab_hbm, out_hbm, idx_v, row_v, out_v, sem):
        nc = mesh.num_cores
        wid = lax.axis_index("s") * nc + lax.axis_index("c")
        r0 = row_lo + wid * rows_per_w

        pending = [None] * n_buf
        for i in range(rows_per_w):
            r = r0 + i
            f = r // d_per_f
            prev_f = (r - 1) // d_per_f

            @pl.when(jnp.logical_or(f != prev_f, i == 0))
            def _load_idx():
                pltpu.sync_copy(idx_hbm.at[f], idx_v)

            pltpu.sync_copy(tab_hbm.at[r], row_v)

            for c in range(n_och):
                b = (i * n_och + c) % n_buf
                if pending[b] is not None:
                    pending[b].wait()
                cbase = c * och

                @plsc.parallel_loop(0, och // 16, step=1, unroll=unroll)
                def _gather(j, cbase=cbase, b=b):
                    o = j * 16
                    iv = idx_v[pl.ds(cbase + o, 16)]
                    out_v[b, pl.ds(o, 16)] = plsc.load_gather(row_v, [iv])
                pending[b] = pltpu.async_copy(
                    out_v.at[b],
                    out_hbm.at[r - row_lo, pl.ds(cbase, och)], sem)
        for p in pending:
            if p is not None:
                p.wait()

    return sel_kernel(idx_t, tab_t)


def _mlp1_body(xt_ref, w1_ref, h_ref):
    dn = (((0,), (0,)), ((), ()))
    h_ref[...] = lax.dot_general(xt_ref[...], w1_ref[...], dn,
                                 preferred_element_type=jnp.float32)


def _tc_mlp1(x_t1, W1a1, block_b):
    R1, B = x_t1.shape
    H = W1a1.shape[1]
    return pl.pallas_call(
        _mlp1_body,
        grid=(B // block_b,),
        in_specs=[
            pl.BlockSpec((R1, block_b), lambda i: (0, i)),
            pl.BlockSpec((R1, H), lambda i: (0, 0)),
        ],
        out_specs=pl.BlockSpec((block_b, H), lambda i: (i, 0)),
        out_shape=jax.ShapeDtypeStruct((B, H), jnp.float32),
    )(x_t1, W1a1)


def _mlp2_body(xt_ref, h1_ref, numt_ref, w1_ref, b1_ref, w2_ref, b2_ref,
               out_ref, *, R2):
    dn = (((0,), (0,)), ((), ()))
    w1 = w1_ref[...]
    h = h1_ref[...] + lax.dot_general(xt_ref[...], w1[:R2], dn,
                                      preferred_element_type=jnp.float32)
    h += lax.dot_general(numt_ref[...], w1[R2:], dn,
                         preferred_element_type=jnp.float32)
    h += b1_ref[...]
    h = jnp.maximum(h, 0.0)
    y = jnp.dot(h, w2_ref[...], preferred_element_type=jnp.float32)
    y += b2_ref[...]
    out_ref[...] = jax.nn.sigmoid(y)


def _tc_mlp2(x_t2, h1, num_t, W1bc, b1, W2, b2, block_b):
    R2, B = x_t2.shape
    NUM = num_t.shape[0]
    K, H = W1bc.shape
    OUT = W2.shape[1]
    return pl.pallas_call(
        functools.partial(_mlp2_body, R2=R2),
        grid=(B // block_b,),
        in_specs=[
            pl.BlockSpec((R2, block_b), lambda i: (0, i)),
            pl.BlockSpec((block_b, H), lambda i: (i, 0)),
            pl.BlockSpec((NUM, block_b), lambda i: (0, i)),
            pl.BlockSpec((K, H), lambda i: (0, 0)),
            pl.BlockSpec((1, H), lambda i: (0, 0)),
            pl.BlockSpec((H, OUT), lambda i: (0, 0)),
            pl.BlockSpec((1, OUT), lambda i: (0, 0)),
        ],
        out_specs=pl.BlockSpec((block_b, OUT), lambda i: (i, 0)),
        out_shape=jax.ShapeDtypeStruct((B, OUT), jnp.float32),
    )(x_t2, h1, num_t, W1bc, b1.reshape(1, H), W2, b2.reshape(1, OUT))


def kernel(categorical_inputs, numerical_inputs, tables, W1, b1, W2, b2):
    B, F = categorical_inputs.shape
    _, V, D = tables.shape
    R = F * D
    R1 = (F // 2) * D

    idx_t = categorical_inputs.astype(jnp.int32).T
    tab_t = tables.transpose(0, 2, 1).reshape(R, V)

    x_t1 = _sc_select_gather(idx_t, tab_t, B, V, 0, R1)
    x_t2 = _sc_select_gather(idx_t, tab_t, B, V, R1, R - R1)

    h1 = _tc_mlp1(x_t1, W1[:R1], block_b=4096)
    num_t = numerical_inputs.T
    return _tc_mlp2(x_t2, h1, num_t, W1[R1:], b1, W2, b2, block_b=4096)


# speedup vs baseline: 1.0258x; 1.0258x over previous
import functools

import jax
import jax.numpy as jnp
from jax import lax
from jax.experimental import pallas as pl
from jax.experimental.pallas import tpu as pltpu
from jax.experimental.pallas import tpu_sc as plsc


def _sc_select_gather(idx_t, tab_t, B, V):
    R = tab_t.shape[0]
    F = idx_t.shape[0]
    d_per_f = R // F
    n_workers = 32
    rows_per_w = R // n_workers
    och = 4096
    n_och = B // och
    n_buf = 2
    mesh = plsc.VectorSubcoreMesh(core_axis_name="c", subcore_axis_name="s")

    unroll = 8

    @functools.partial(
        pl.kernel,
        out_type=jax.ShapeDtypeStruct((R, B), jnp.float32),
        mesh=mesh,
        scratch_types=[
            pltpu.VMEM((B,), jnp.int32),
            pltpu.VMEM((V,), jnp.float32),
            pltpu.VMEM((n_buf, och), jnp.float32),
            pltpu.SemaphoreType.DMA,
        ],
        compiler_params=pltpu.CompilerParams(use_tc_tiling_on_sc=True,
                                             needs_layout_passes=False),
    )
    def sel_kernel(idx_hbm, tab_hbm, out_hbm, idx_v, row_v, out_v, sem):
        nc = mesh.num_cores
        wid = lax.axis_index("s") * nc + lax.axis_index("c")
        r0 = wid * rows_per_w

        pending = [None] * n_buf
        for i in range(rows_per_w):
            r = r0 + i
            f = r // d_per_f
            prev_f = (r - 1) // d_per_f

            @pl.when(jnp.logical_or(f != prev_f, i == 0))
            def _load_idx():
                pltpu.sync_copy(idx_hbm.at[f], idx_v)

            pltpu.sync_copy(tab_hbm.at[r], row_v)

            for c in range(n_och):
                b = (i * n_och + c) % n_buf
                if pending[b] is not None:
                    pending[b].wait()
                cbase = c * och

                @plsc.parallel_loop(0, och // 16, step=1, unroll=unroll)
                def _gather(j, cbase=cbase, b=b):
                    o = j * 16
                    iv = idx_v[pl.ds(cbase + o, 16)]
                    out_v[b, pl.ds(o, 16)] = plsc.load_gather(row_v, [iv])
                pending[b] = pltpu.async_copy(
                    out_v.at[b], out_hbm.at[r, pl.ds(cbase, och)], sem)
        for p in pending:
            if p is not None:
                p.wait()

    return sel_kernel(idx_t, tab_t)


def _mlp_body(R, xt_ref, numt_ref, w1_ref, b1_ref, w2_ref, b2_ref, out_ref):
    dn = (((0,), (0,)), ((), ()))
    w1 = w1_ref[...]
    h = lax.dot_general(xt_ref[...], w1[:R], dn,
                        preferred_element_type=jnp.float32)
    h += lax.dot_general(numt_ref[...], w1[R:], dn,
                         preferred_element_type=jnp.float32)
    h += b1_ref[...]
    h = jnp.maximum(h, 0.0)
    y = jnp.dot(h, w2_ref[...], preferred_element_type=jnp.float32)
    y += b2_ref[...]
    out_ref[...] = jax.nn.sigmoid(y)


def _tc_mlp(x_t, num_t, W1, b1, W2, b2, block_b):
    R, B = x_t.shape
    NUM = num_t.shape[0]
    K, H = W1.shape
    OUT = W2.shape[1]
    grid = (B // block_b,)
    return pl.pallas_call(
        functools.partial(_mlp_body, R),
        grid=grid,
        in_specs=[
            pl.BlockSpec((R, block_b), lambda i: (0, i)),
            pl.BlockSpec((NUM, block_b), lambda i: (0, i)),
            pl.BlockSpec((K, H), lambda i: (0, 0)),
            pl.BlockSpec((1, H), lambda i: (0, 0)),
            pl.BlockSpec((H, OUT), lambda i: (0, 0)),
            pl.BlockSpec((1, OUT), lambda i: (0, 0)),
        ],
        out_specs=pl.BlockSpec((block_b, OUT), lambda i: (i, 0)),
        out_shape=jax.ShapeDtypeStruct((B, OUT), jnp.float32),
    )(x_t, num_t, W1, b1.reshape(1, H), W2, b2.reshape(1, OUT))


def kernel(categorical_inputs, numerical_inputs, tables, W1, b1, W2, b2):
    B, F = categorical_inputs.shape
    _, V, D = tables.shape

    idx_t = categorical_inputs.astype(jnp.int32).T
    tab_t = tables.transpose(0, 2, 1).reshape(F * D, V)

    x_t = _sc_select_gather(idx_t, tab_t, B, V)

    num_t = numerical_inputs.T
    return _tc_mlp(x_t, num_t, W1, b1, W2, b2, block_b=4096)
